# Initial kernel scaffold; baseline (speedup 1.0000x reference)
import jax
import jax.numpy as jnp
from jax.experimental import pallas as pl


def kernel(x, w1, b1, g1, be1, w2, b2, g2, be2, w3, b3, g3, be3, w4, b4):
    raise NotImplementedError("write your pallas kernel here")


# speedup vs baseline: 2.2624x; 2.2624x over previous
import jax
import jax.numpy as jnp
from jax.experimental import pallas as pl
from jax.experimental.pallas import tpu as pltpu

EPS = 1e-5


def _conv_kernel(xt_ref, w1t_ref, s1_ref, t1_ref, w2t_ref, s2_ref, t2_ref,
                 o_ref):
    L = o_ref.shape[2]
    C2 = o_ref.shape[1]

    h1 = jnp.dot(w1t_ref[...], xt_ref[0], preferred_element_type=jnp.float32)
    h1 = jnp.maximum(h1 * s1_ref[...] + t1_ref[...], 0.0)

    p = jnp.dot(w2t_ref[...], h1, preferred_element_type=jnp.float32)
    p0 = p[0:C2]
    p1 = p[C2:2 * C2]
    p2 = p[2 * C2:3 * C2]
    z = jnp.zeros((C2, 1), jnp.float32)
    acc = p1
    acc = acc + jnp.concatenate([z, p0[:, :L - 1]], axis=1)
    acc = acc + jnp.concatenate([p2[:, 1:], z], axis=1)
    o = jnp.maximum(acc * s2_ref[...] + t2_ref[...], 0.0)
    o_ref[0] = o.astype(jnp.bfloat16)


def _fc1_kernel(x_ref, w3_ref, o_ref, acc_ref):
    k = pl.program_id(1)

    @pl.when(k == 0)
    def _():
        acc_ref[...] = jnp.zeros_like(acc_ref)

    acc_ref[...] += jax.lax.dot_general(
        x_ref[...], w3_ref[...].astype(jnp.bfloat16),
        dimension_numbers=(((1,), (1,)), ((), ())),
        preferred_element_type=jnp.float32)

    @pl.when(k == pl.num_programs(1) - 1)
    def _():
        o_ref[0] = acc_ref[...]


def _head_kernel(p_ref, s3_ref, t3_ref, w4_ref, b4_ref, o_ref):
    h = p_ref[0] + p_ref[1]
    h3 = jnp.maximum(h * s3_ref[...] + t3_ref[...], 0.0)
    y = jnp.dot(h3, w4_ref[...], preferred_element_type=jnp.float32)
    o_ref[...] = jnp.maximum(y + b4_ref[...], 0.0)


def _affine_cols(bias, gamma, beta):
    scale = gamma / jnp.sqrt(1.0 + EPS)
    shift = bias * scale + beta
    return scale.reshape(-1, 1), shift.reshape(-1, 1)


def kernel(x, w1, b1, g1, be1, w2, b2, g2, be2, w3, b3, g3, be3, w4, b4):
    B, Cin, L = x.shape
    C1 = w1.shape[0]
    C2 = w2.shape[0]
    H = w3.shape[0]
    CLS = w4.shape[0]

    s1, t1 = _affine_cols(b1, g1, be1)
    s2, t2 = _affine_cols(b2, g2, be2)
    s3c = (g3 / jnp.sqrt(1.0 + EPS)).reshape(1, -1)
    t3c = (b3 * s3c[0] + be3).reshape(1, -1)

    w1t = jnp.transpose(w1, (2, 1, 0)).reshape(3 * Cin, C1).T
    w2t = jnp.transpose(w2, (2, 0, 1)).reshape(3 * C2, C1)

    x_pad = jnp.pad(x, ((0, 0), (0, 0), (1, 1)))
    xt = jnp.stack([x_pad[:, 0, 0:L], x_pad[:, 1, 0:L],
                    x_pad[:, 0, 1:L + 1], x_pad[:, 1, 1:L + 1],
                    x_pad[:, 0, 2:L + 2], x_pad[:, 1, 2:L + 2]],
                   axis=1)

    h2t = pl.pallas_call(
        _conv_kernel,
        out_shape=jax.ShapeDtypeStruct((B, C2, L), jnp.bfloat16),
        grid=(B,),
        in_specs=[
            pl.BlockSpec((1, 3 * Cin, L), lambda b: (b, 0, 0)),
            pl.BlockSpec((C1, 3 * Cin), lambda b: (0, 0)),
            pl.BlockSpec((C1, 1), lambda b: (0, 0)),
            pl.BlockSpec((C1, 1), lambda b: (0, 0)),
            pl.BlockSpec((3 * C2, C1), lambda b: (0, 0)),
            pl.BlockSpec((C2, 1), lambda b: (0, 0)),
            pl.BlockSpec((C2, 1), lambda b: (0, 0)),
        ],
        out_specs=pl.BlockSpec((1, C2, L), lambda b: (b, 0, 0)),
        compiler_params=pltpu.CompilerParams(
            dimension_semantics=("parallel",)),
    )(xt, w1t, s1, t1, w2t, s2, t2)

    K = C2 * L
    flat = h2t.reshape(B, K)
    TK = 8192
    nk = K // TK

    partials = pl.pallas_call(
        _fc1_kernel,
        out_shape=jax.ShapeDtypeStruct((2, B, H), jnp.float32),
        grid=(2, nk // 2),
        in_specs=[
            pl.BlockSpec((B, TK), lambda c, k: (0, c * (nk // 2) + k)),
            pl.BlockSpec((H, TK), lambda c, k: (0, c * (nk // 2) + k)),
        ],
        out_specs=pl.BlockSpec((1, B, H), lambda c, k: (c, 0, 0)),
        scratch_shapes=[pltpu.VMEM((B, H), jnp.float32)],
        compiler_params=pltpu.CompilerParams(
            dimension_semantics=("parallel", "arbitrary")),
    )(flat, w3)

    return pl.pallas_call(
        _head_kernel,
        out_shape=jax.ShapeDtypeStruct((B, CLS), jnp.float32),
        in_specs=[
            pl.BlockSpec((2, B, H), lambda: (0, 0, 0)),
            pl.BlockSpec((1, H), lambda: (0, 0)),
            pl.BlockSpec((1, H), lambda: (0, 0)),
            pl.BlockSpec((H, CLS), lambda: (0, 0)),
            pl.BlockSpec((1, CLS), lambda: (0, 0)),
        ],
        out_specs=pl.BlockSpec((B, CLS), lambda: (0, 0)),
    )(partials, s3c, t3c, w4.T, b4.reshape(1, -1))


# speedup vs baseline: 3.1874x; 1.4089x over previous
import jax
import jax.numpy as jnp
from jax.experimental import pallas as pl
from jax.experimental.pallas import tpu as pltpu

EPS = 1e-5
EB = 8


def _conv_kernel(x_ref, w1t_ref, s1_ref, t1_ref, w2t_ref, s2_ref, t2_ref,
                 o_ref):
    L = o_ref.shape[2]
    C2 = o_ref.shape[0]

    for e in range(EB):
        xe = x_ref[e]
        z2 = jnp.zeros((2, 1), jnp.float32)
        xt = jnp.concatenate([
            jnp.concatenate([z2, xe[:, :L - 1]], axis=1),
            xe,
            jnp.concatenate([xe[:, 1:], z2], axis=1),
        ], axis=0)

        h1 = jnp.dot(w1t_ref[...], xt, preferred_element_type=jnp.float32)
        h1 = jnp.maximum(h1 * s1_ref[...] + t1_ref[...], 0.0)

        p = jnp.dot(w2t_ref[...], h1, preferred_element_type=jnp.float32)
        p0 = p[0:C2]
        p1 = p[C2:2 * C2]
        p2 = p[2 * C2:3 * C2]
        z = jnp.zeros((C2, 1), jnp.float32)
        acc = p1
        acc = acc + jnp.concatenate([z, p0[:, :L - 1]], axis=1)
        acc = acc + jnp.concatenate([p2[:, 1:], z], axis=1)
        o = jnp.maximum(acc * s2_ref[...] + t2_ref[...], 0.0)
        o_ref[:, e, :] = o.astype(jnp.bfloat16)


def _fc1_kernel(x_ref, w3_ref, o_ref, acc_ref):
    k = pl.program_id(1)
    L = x_ref.shape[2]
    TC = x_ref.shape[0]

    @pl.when(k == 0)
    def _():
        acc_ref[...] = jnp.zeros_like(acc_ref)

    for c in range(TC):
        acc_ref[...] += jax.lax.dot_general(
            x_ref[c], w3_ref[:, c * L:(c + 1) * L].astype(jnp.bfloat16),
            dimension_numbers=(((1,), (1,)), ((), ())),
            preferred_element_type=jnp.float32)

    @pl.when(k == pl.num_programs(1) - 1)
    def _():
        o_ref[0] = acc_ref[...]


def _head_kernel(p_ref, s3_ref, t3_ref, w4_ref, b4_ref, o_ref):
    h = p_ref[0] + p_ref[1]
    h3 = jnp.maximum(h * s3_ref[...] + t3_ref[...], 0.0)
    y = jnp.dot(h3, w4_ref[...], preferred_element_type=jnp.float32)
    o_ref[...] = jnp.maximum(y + b4_ref[...], 0.0)


def _affine_cols(bias, gamma, beta):
    scale = gamma / jnp.sqrt(1.0 + EPS)
    shift = bias * scale + beta
    return scale.reshape(-1, 1), shift.reshape(-1, 1)


def kernel(x, w1, b1, g1, be1, w2, b2, g2, be2, w3, b3, g3, be3, w4, b4):
    B, Cin, L = x.shape
    C1 = w1.shape[0]
    C2 = w2.shape[0]
    H = w3.shape[0]
    CLS = w4.shape[0]

    s1, t1 = _affine_cols(b1, g1, be1)
    s2, t2 = _affine_cols(b2, g2, be2)
    s3c = (g3 / jnp.sqrt(1.0 + EPS)).reshape(1, -1)
    t3c = (b3 * s3c[0] + be3).reshape(1, -1)

    w1t = jnp.transpose(w1, (2, 1, 0)).reshape(3 * Cin, C1).T
    w2t = jnp.transpose(w2, (2, 0, 1)).reshape(3 * C2, C1)

    h2t = pl.pallas_call(
        _conv_kernel,
        out_shape=jax.ShapeDtypeStruct((C2, B, L), jnp.bfloat16),
        grid=(B // EB,),
        in_specs=[
            pl.BlockSpec((EB, Cin, L), lambda b: (b, 0, 0)),
            pl.BlockSpec((C1, 3 * Cin), lambda b: (0, 0)),
            pl.BlockSpec((C1, 1), lambda b: (0, 0)),
            pl.BlockSpec((C1, 1), lambda b: (0, 0)),
            pl.BlockSpec((3 * C2, C1), lambda b: (0, 0)),
            pl.BlockSpec((C2, 1), lambda b: (0, 0)),
            pl.BlockSpec((C2, 1), lambda b: (0, 0)),
        ],
        out_specs=pl.BlockSpec((C2, EB, L), lambda b: (0, b, 0)),
        compiler_params=pltpu.CompilerParams(
            dimension_semantics=("parallel",)),
    )(x, w1t, s1, t1, w2t, s2, t2)

    TC = 8
    nk = C2 // TC

    partials = pl.pallas_call(
        _fc1_kernel,
        out_shape=jax.ShapeDtypeStruct((2, B, H), jnp.float32),
        grid=(2, nk // 2),
        in_specs=[
            pl.BlockSpec((TC, B, L), lambda c, k: (c * (nk // 2) + k, 0, 0)),
            pl.BlockSpec((H, TC * L), lambda c, k: (0, c * (nk // 2) + k)),
        ],
        out_specs=pl.BlockSpec((1, B, H), lambda c, k: (c, 0, 0)),
        scratch_shapes=[pltpu.VMEM((B, H), jnp.float32)],
        compiler_params=pltpu.CompilerParams(
            dimension_semantics=("parallel", "arbitrary")),
    )(h2t, w3)

    return pl.pallas_call(
        _head_kernel,
        out_shape=jax.ShapeDtypeStruct((B, CLS), jnp.float32),
        in_specs=[
            pl.BlockSpec((2, B, H), lambda: (0, 0, 0)),
            pl.BlockSpec((1, H), lambda: (0, 0)),
            pl.BlockSpec((1, H), lambda: (0, 0)),
            pl.BlockSpec((H, CLS), lambda: (0, 0)),
            pl.BlockSpec((1, CLS), lambda: (0, 0)),
        ],
        out_specs=pl.BlockSpec((B, CLS), lambda: (0, 0)),
    )(partials, s3c, t3c, w4.T, b4.reshape(1, -1))
